# Initial kernel scaffold; baseline (speedup 1.0000x reference)
import jax
import jax.numpy as jnp
from jax.experimental import pallas as pl


def kernel(x, emb_table, W_out, b_out):
    raise NotImplementedError("write your pallas kernel here")


# speedup vs baseline: 2.2298x; 2.2298x over previous
import functools

import jax
import jax.numpy as jnp
import numpy as np
from jax import lax
from jax.experimental import pallas as pl
from jax.experimental.pallas import tpu as pltpu
from jax.experimental.pallas import tpu_sc as plsc

_VOCAB = 100000
_D = 512
_NTOK = 100
_NPAD = 128
_B = 1024
_L = 200

_NW = 32
_ROWS_PER_W = _B // _NW
_ROW_BLK = 2000
_GRID = _VOCAB // _ROW_BLK


def _sin_pe(seq_len, d_model):
    pos = np.arange(seq_len, dtype=np.float32)[:, None]
    div = np.exp(np.arange(0, d_model, 2, dtype=np.float32)
                 * (-np.log(10000.0) / d_model))
    pe = np.zeros((seq_len, d_model), dtype=np.float32)
    pe[:, 0::2] = np.sin(pos * div)
    pe[:, 1::2] = np.cos(pos * div)
    return pe


_PE = _sin_pe(_L, _D)


def _proj_body(emb_ref, wt_ref, pe_ref, b_ref, p_ref, pep_ref):
    p_ref[...] = jnp.dot(emb_ref[...], wt_ref[...],
                         preferred_element_type=jnp.float32)

    @pl.when(pl.program_id(0) == 0)
    def _():
        pep_ref[...] = (jnp.dot(pe_ref[...], wt_ref[...],
                                preferred_element_type=jnp.float32)
                        + b_ref[...])


def _project_table(emb_table, wt, b2d):
    return pl.pallas_call(
        _proj_body,
        grid=(_GRID,),
        in_specs=[
            pl.BlockSpec((_ROW_BLK, _D), lambda i: (i, 0)),
            pl.BlockSpec((_D, _NPAD), lambda i: (0, 0)),
            pl.BlockSpec((_L, _D), lambda i: (0, 0)),
            pl.BlockSpec((1, _NPAD), lambda i: (0, 0)),
        ],
        out_specs=[
            pl.BlockSpec((_ROW_BLK, _NPAD), lambda i: (i, 0)),
            pl.BlockSpec((_L, _NPAD), lambda i: (0, 0)),
        ],
        out_shape=[
            jax.ShapeDtypeStruct((_VOCAB, _NPAD), jnp.float32),
            jax.ShapeDtypeStruct((_L, _NPAD), jnp.float32),
        ],
    )(emb_table, wt, jnp.asarray(_PE), b2d)


def _gather_body(p_hbm, x_hbm, pep_hbm, out_hbm, idx_v, rows_v, pep_v, sem):
    wid = lax.axis_index("s") * 2 + lax.axis_index("c")
    pltpu.sync_copy(pep_hbm, pep_v)

    def per_row(r, carry):
        row = wid * _ROWS_PER_W + r
        pltpu.sync_copy(x_hbm.at[row], idx_v)
        d0 = pltpu.async_copy(p_hbm.at[idx_v.at[0]],
                              rows_v.at[pl.ds(0, _L // 2)], sem)
        d1 = pltpu.async_copy(p_hbm.at[idx_v.at[1]],
                              rows_v.at[pl.ds(_L // 2, _L // 2)], sem)
        d0.wait()
        d1.wait()

        def add_pe(j, c2):
            for c in range(_NPAD // 16):
                v = pep_v[j, pl.ds(c * 16, 16)]
                plsc.addupdate(rows_v.at[j, pl.ds(c * 16, 16)], v)
            return c2

        lax.fori_loop(0, _L, add_pe, 0, unroll=2)
        pltpu.sync_copy(rows_v, out_hbm.at[row])
        return carry

    lax.fori_loop(0, _ROWS_PER_W, per_row, 0)


_gather_pe = functools.partial(
    pl.kernel,
    out_type=jax.ShapeDtypeStruct((_B, _L, _NPAD), jnp.float32),
    mesh=plsc.VectorSubcoreMesh(core_axis_name="c", subcore_axis_name="s",
                                num_cores=2, num_subcores=16),
    scratch_types=[
        pltpu.VMEM((2, _L // 2), jnp.int32),
        pltpu.VMEM((_L, _NPAD), jnp.float32),
        pltpu.VMEM((_L, _NPAD), jnp.float32),
        pltpu.SemaphoreType.DMA,
    ],
)(_gather_body)


def kernel(x, emb_table, W_out, b_out):
    wt = jnp.pad(W_out, ((0, _NPAD - _NTOK), (0, 0))).T
    b2d = jnp.pad(b_out, (0, _NPAD - _NTOK)).reshape(1, _NPAD)
    p, pep = _project_table(emb_table, wt, b2d)
    out = _gather_pe(p, x.reshape(_B, 2, _L // 2), pep)
    return out[:, :, :_NTOK]
